# baseline (speedup 1.0000x reference)
import functools

import jax
import jax.numpy as jnp
from jax import lax
from jax.experimental import pallas as pl
from jax.experimental.pallas import tpu as pltpu
from jax.experimental.pallas import tpu_sc as plsc

N_NODES = 10000
N_EDGES = 320000
D = 128

NC = 2
NS = 16
NW = NC * NS
EPT = N_EDGES // NW
N_PAD = 10240
ROWS_PT = N_PAD // NS

PW = 5120
NPASS = 2


def _sc_degree_body(dst_hbm, deg_part_hbm, idx_v, hist_v, res_v):
    c = lax.axis_index("c")
    s = lax.axis_index("s")
    wid = s * NC + c
    base = wid * EPT
    pltpu.sync_copy(dst_hbm.at[pl.ds(base, EPT)], idx_v)

    lanes = jnp.arange(16, dtype=jnp.int32)
    ones = jnp.ones((16,), jnp.float32)
    zeros = jnp.zeros((16,), jnp.float32)

    for p in range(NPASS):
        lo = p * PW

        @pl.loop(0, 16 * PW // 16)
        def _zero(col):
            hist_v[pl.ds(col * 16, 16)] = zeros

        @pl.loop(0, EPT // 16)
        def _scan(e):
            idx16 = idx_v[pl.ds(e * 16, 16)]
            local = idx16 - lo
            mask = (local >= 0) & (local < PW)
            localc = jnp.where(mask, local, 0)
            plsc.addupdate_scatter(hist_v, [lanes * PW + localc], ones,
                                   mask=mask)

        @pl.loop(0, PW // 16)
        def _reduce(col):
            acc = hist_v[pl.ds(col * 16, 16)]
            for r in range(1, 16):
                acc = acc + hist_v[pl.ds(r * PW + col * 16, 16)]
            res_v[pl.ds(col * 16, 16)] = acc

        pltpu.sync_copy(res_v, deg_part_hbm.at[wid, pl.ds(lo, PW)])


_sc_degree = functools.partial(
    pl.kernel,
    out_type=jax.ShapeDtypeStruct((NW, NPASS * PW), jnp.float32),
    mesh=plsc.VectorSubcoreMesh(core_axis_name="c", subcore_axis_name="s",
                                num_cores=NC, num_subcores=NS),
    scratch_types=[
        pltpu.VMEM((EPT,), jnp.int32),
        pltpu.VMEM((16 * PW,), jnp.float32),
        pltpu.VMEM((PW,), jnp.float32),
    ],
    compiler_params=pltpu.CompilerParams(needs_layout_passes=False),
)(_sc_degree_body)


def _tc_prep_body(deg_ref, data_ref, w_ref, y_ref, dinv_ref):
    deg = jnp.sum(deg_ref[...], axis=0)[:N_NODES] + 1.0
    dinv = lax.rsqrt(deg)[:, None]
    xw = jnp.dot(data_ref[...], w_ref[...],
                 preferred_element_type=jnp.float32)
    y_ref[...] = xw * dinv
    dinv_ref[...] = dinv


def _tc_prep(deg_part, data, W):
    return pl.pallas_call(
        _tc_prep_body,
        out_shape=[
            jax.ShapeDtypeStruct((N_NODES, D), jnp.float32),
            jax.ShapeDtypeStruct((N_NODES, 1), jnp.float32),
        ],
    )(deg_part, data, W)


G = 64
EPT_P = 10240
NCHUNK = EPT_P // G
E_PAD = NW * EPT_P


def _sc_scatter_body(src_hbm, dst2_hbm, y_hbm, zeros_hbm, part_hbm,
                     sidx_v, didx_v, rows_a, rows_b, sem_a, sem_b, acc_sh):
    c = lax.axis_index("c")
    s = lax.axis_index("s")
    wid = s * NC + c
    base = wid * EPT_P

    pltpu.sync_copy(zeros_hbm, acc_sh.at[pl.ds(s * ROWS_PT, ROWS_PT), :])
    pltpu.sync_copy(src_hbm.at[pl.ds(base, EPT_P)],
                    sidx_v.at[pl.ds(0, EPT_P)])
    zeros16 = jnp.zeros((16,), jnp.int32)
    for j in range(G // 16):
        sidx_v[pl.ds(EPT_P + j * 16, 16)] = zeros16
    pltpu.sync_copy(dst2_hbm.at[pl.ds(wid * NCHUNK, NCHUNK), :], didx_v)

    plsc.subcore_barrier()

    def start(i, rows, sem):
        pltpu.async_copy(y_hbm.at[sidx_v.at[pl.ds(i * G, G)]], rows, sem)

    def wait(rows, sem):
        pltpu.make_async_copy(y_hbm.at[sidx_v.at[pl.ds(0, G)]], rows,
                              sem).wait()

    start(0, rows_a, sem_a)

    @pl.loop(0, NCHUNK, step=2)
    def _chunk(i):
        start(i + 1, rows_b, sem_b)
        wait(rows_a, sem_a)
        pltpu.sync_copy(rows_a, acc_sh.at[didx_v.at[i]], add=True)
        start(i + 2, rows_a, sem_a)
        wait(rows_b, sem_b)
        pltpu.sync_copy(rows_b, acc_sh.at[didx_v.at[i + 1]], add=True)

    wait(rows_a, sem_a)

    plsc.subcore_barrier()

    pltpu.sync_copy(acc_sh.at[pl.ds(s * ROWS_PT, ROWS_PT), :],
                    part_hbm.at[c, pl.ds(s * ROWS_PT, ROWS_PT), :])


_sc_scatter = functools.partial(
    pl.kernel,
    out_type=jax.ShapeDtypeStruct((NC, N_PAD, D), jnp.float32),
    mesh=plsc.VectorSubcoreMesh(core_axis_name="c", subcore_axis_name="s",
                                num_cores=NC, num_subcores=NS),
    scratch_types=[
        pltpu.VMEM((EPT_P + G,), jnp.int32),
        pltpu.VMEM((NCHUNK, G), jnp.int32),
        pltpu.VMEM((G, D), jnp.float32),
        pltpu.VMEM((G, D), jnp.float32),
        pltpu.SemaphoreType.DMA,
        pltpu.SemaphoreType.DMA,
        pltpu.VMEM_SHARED((N_PAD, D), jnp.float32),
    ],
    compiler_params=pltpu.CompilerParams(needs_layout_passes=False),
)(_sc_scatter_body)


def _tc_finish_body(part_ref, y_ref, dinv_ref, b_ref, g_ref, beta_ref,
                    o_ref):
    s = part_ref[0, :N_NODES] + part_ref[1, :N_NODES] + y_ref[...]
    pre = s * dinv_ref[...] + b_ref[...]
    r = jnp.maximum(pre, 0.0)
    mean = jnp.mean(r, axis=0, keepdims=True)
    var = jnp.mean((r - mean) ** 2, axis=0, keepdims=True)
    o_ref[...] = (r - mean) / jnp.sqrt(var + 1e-5) * g_ref[...] + beta_ref[...]


def _tc_finish(part, y, dinv, b, g, beta):
    return pl.pallas_call(
        _tc_finish_body,
        out_shape=jax.ShapeDtypeStruct((N_NODES, D), jnp.float32),
    )(part, y, dinv, b, g, beta)


def kernel(data, edge_index, W, b, bn_gamma, bn_beta):
    src = edge_index[0].astype(jnp.int32)
    dst = edge_index[1].astype(jnp.int32)
    deg_part = _sc_degree(dst)
    y, dinv = _tc_prep(deg_part, data, W)
    zeros_rows = jnp.zeros((ROWS_PT, D), jnp.float32)
    npad = E_PAD - N_EDGES
    src_p = jnp.concatenate([src, jnp.zeros((npad,), jnp.int32)])
    dst2 = jnp.concatenate(
        [dst, jnp.full((npad,), N_NODES, jnp.int32)]).reshape(-1, G)
    part = _sc_scatter(src_p, dst2, y, zeros_rows)
    return _tc_finish(part, y, dinv, b.reshape(1, D),
                      bn_gamma.reshape(1, D), bn_beta.reshape(1, D))


# speedup vs baseline: 3.8549x; 3.8549x over previous
import functools

import jax
import jax.numpy as jnp
from jax import lax
from jax.experimental import pallas as pl
from jax.experimental.pallas import tpu as pltpu
from jax.experimental.pallas import tpu_sc as plsc

N_NODES = 10000
N_EDGES = 320000
D = 128

NC = 2
NS = 16
NW = NC * NS
EPT = N_EDGES // NW
N_PAD = 10240
ROWS_PT = N_PAD // NS

def _sc_degree_body(dst_hbm, deg_part_hbm, idx_v, hist_v):
    c = lax.axis_index("c")
    s = lax.axis_index("s")
    wid = s * NC + c
    pltpu.sync_copy(dst_hbm.at[pl.ds(wid * EPT_P, EPT_P)], idx_v)

    ones = jnp.ones((16,), jnp.float32)
    zeros = jnp.zeros((16,), jnp.float32)

    @pl.loop(0, N_PAD // 16)
    def _zero(col):
        hist_v[pl.ds(col * 16, 16)] = zeros

    @pl.loop(0, EPT_P // 16)
    def _scan(e):
        idx16 = idx_v[pl.ds(e * 16, 16)]
        plsc.addupdate_scatter(hist_v, [idx16], ones)

    pltpu.sync_copy(hist_v, deg_part_hbm.at[wid])


_sc_degree = functools.partial(
    pl.kernel,
    out_type=jax.ShapeDtypeStruct((NW, 10240), jnp.float32),
    mesh=plsc.VectorSubcoreMesh(core_axis_name="c", subcore_axis_name="s",
                                num_cores=NC, num_subcores=NS),
    scratch_types=[
        pltpu.VMEM((10240,), jnp.int32),
        pltpu.VMEM((10240,), jnp.float32),
    ],
    compiler_params=pltpu.CompilerParams(needs_layout_passes=False),
)(_sc_degree_body)


def _tc_prep_body(deg_ref, data_ref, w_ref, y_ref, dinv_ref):
    deg = jnp.sum(deg_ref[...], axis=0)[:N_NODES] + 1.0
    dinv = lax.rsqrt(deg)[:, None]
    xw = jnp.dot(data_ref[...], w_ref[...],
                 preferred_element_type=jnp.float32)
    y_ref[...] = xw * dinv
    dinv_ref[...] = dinv


def _tc_prep(deg_part, data, W):
    return pl.pallas_call(
        _tc_prep_body,
        out_shape=[
            jax.ShapeDtypeStruct((N_NODES, D), jnp.float32),
            jax.ShapeDtypeStruct((N_NODES, 1), jnp.float32),
        ],
    )(deg_part, data, W)


G = 128
EPT_P = 10240
NCHUNK = EPT_P // G
E_PAD = NW * EPT_P


def _sc_scatter_body(src_hbm, dst_hbm, y_hbm, zeros_hbm, part_hbm,
                     sidx_a, sidx_b, didx_a, didx_b, rows_a, rows_b,
                     si_a, si_b, sd_a, sd_b, sg_a, sg_b, acc_sh):
    c = lax.axis_index("c")
    s = lax.axis_index("s")
    wid = s * NC + c
    base = wid * EPT_P

    pltpu.sync_copy(zeros_hbm, acc_sh.at[pl.ds(s * ROWS_PT, ROWS_PT), :])

    plsc.subcore_barrier()

    def start_i(i, sidx, didx, si, sd):
        pltpu.async_copy(src_hbm.at[pl.ds(base + i * G, G)], sidx, si)
        pltpu.async_copy(dst_hbm.at[pl.ds(base + i * G, G)], didx, sd)

    def wait_i(sidx, didx, si, sd):
        pltpu.make_async_copy(src_hbm.at[pl.ds(0, G)], sidx, si).wait()
        pltpu.make_async_copy(dst_hbm.at[pl.ds(0, G)], didx, sd).wait()

    def start_g(sidx, rows, sg):
        pltpu.async_copy(y_hbm.at[sidx], rows, sg)

    def wait_g(sidx, rows, sg):
        pltpu.make_async_copy(y_hbm.at[sidx], rows, sg).wait()

    start_i(0, sidx_a, didx_a, si_a, sd_a)
    start_i(1, sidx_b, didx_b, si_b, sd_b)
    wait_i(sidx_a, didx_a, si_a, sd_a)
    start_g(sidx_a, rows_a, sg_a)

    @pl.loop(0, NCHUNK, step=2)
    def _chunk(i):
        wait_i(sidx_b, didx_b, si_b, sd_b)
        start_g(sidx_b, rows_b, sg_b)
        wait_g(sidx_a, rows_a, sg_a)
        pltpu.sync_copy(rows_a, acc_sh.at[didx_a], add=True)
        start_i(i + 2, sidx_a, didx_a, si_a, sd_a)
        wait_i(sidx_a, didx_a, si_a, sd_a)
        start_g(sidx_a, rows_a, sg_a)
        wait_g(sidx_b, rows_b, sg_b)
        pltpu.sync_copy(rows_b, acc_sh.at[didx_b], add=True)
        start_i(i + 3, sidx_b, didx_b, si_b, sd_b)

    wait_g(sidx_a, rows_a, sg_a)
    wait_i(sidx_b, didx_b, si_b, sd_b)

    plsc.subcore_barrier()

    pltpu.sync_copy(acc_sh.at[pl.ds(s * ROWS_PT, ROWS_PT), :],
                    part_hbm.at[c, pl.ds(s * ROWS_PT, ROWS_PT), :])


_sc_scatter = functools.partial(
    pl.kernel,
    out_type=jax.ShapeDtypeStruct((NC, N_PAD, D), jnp.float32),
    mesh=plsc.VectorSubcoreMesh(core_axis_name="c", subcore_axis_name="s",
                                num_cores=NC, num_subcores=NS),
    scratch_types=[
        pltpu.VMEM((G,), jnp.int32),
        pltpu.VMEM((G,), jnp.int32),
        pltpu.VMEM((G,), jnp.int32),
        pltpu.VMEM((G,), jnp.int32),
        pltpu.VMEM((G, D), jnp.float32),
        pltpu.VMEM((G, D), jnp.float32),
        pltpu.SemaphoreType.DMA,
        pltpu.SemaphoreType.DMA,
        pltpu.SemaphoreType.DMA,
        pltpu.SemaphoreType.DMA,
        pltpu.SemaphoreType.DMA,
        pltpu.SemaphoreType.DMA,
        pltpu.VMEM_SHARED((N_PAD, D), jnp.float32),
    ],
)(_sc_scatter_body)


def _tc_finish_body(part_ref, y_ref, dinv_ref, b_ref, g_ref, beta_ref,
                    o_ref):
    s = part_ref[0, :N_NODES] + part_ref[1, :N_NODES] + y_ref[...]
    pre = s * dinv_ref[...] + b_ref[...]
    r = jnp.maximum(pre, 0.0)
    mean = jnp.mean(r, axis=0, keepdims=True)
    var = jnp.mean((r - mean) ** 2, axis=0, keepdims=True)
    o_ref[...] = (r - mean) / jnp.sqrt(var + 1e-5) * g_ref[...] + beta_ref[...]


def _tc_finish(part, y, dinv, b, g, beta):
    return pl.pallas_call(
        _tc_finish_body,
        out_shape=jax.ShapeDtypeStruct((N_NODES, D), jnp.float32),
    )(part, y, dinv, b, g, beta)


def kernel(data, edge_index, W, b, bn_gamma, bn_beta):
    src = edge_index[0].astype(jnp.int32)
    dst = edge_index[1].astype(jnp.int32)
    zeros_rows = jnp.zeros((ROWS_PT, D), jnp.float32)
    ppt = EPT_P - EPT
    pad_src = jnp.broadcast_to(
        (jnp.arange(ppt, dtype=jnp.int32) * 41) % N_NODES, (NW, ppt))
    pad_dst = jnp.broadcast_to(
        N_NODES + (jnp.arange(ppt, dtype=jnp.int32) % (N_PAD - N_NODES)),
        (NW, ppt))
    src_p = jnp.concatenate(
        [src.reshape(NW, EPT), pad_src], axis=1).reshape(-1)
    dst_p = jnp.concatenate(
        [dst.reshape(NW, EPT), pad_dst], axis=1).reshape(-1)
    tail = jnp.zeros((2 * G,), jnp.int32)
    src_p = jnp.concatenate([src_p, tail])
    dst_p = jnp.concatenate([dst_p, N_NODES + tail])
    deg_part = _sc_degree(dst_p)
    y, dinv = _tc_prep(deg_part, data, W)
    part = _sc_scatter(src_p, dst_p, y, zeros_rows)
    return _tc_finish(part, y, dinv, b.reshape(1, D),
                      bn_gamma.reshape(1, D), bn_beta.reshape(1, D))


# speedup vs baseline: 3.8602x; 1.0014x over previous
import functools

import jax
import jax.numpy as jnp
from jax import lax
from jax.experimental import pallas as pl
from jax.experimental.pallas import tpu as pltpu
from jax.experimental.pallas import tpu_sc as plsc

N_NODES = 10000
N_EDGES = 320000
D = 128

NC = 2
NS = 16
NW = NC * NS
EPT = N_EDGES // NW
N_PAD = 10240
ROWS_PT = N_PAD // NS

def _sc_degree_body(dst_hbm, deg_part_hbm, idx_v, hist_v):
    c = lax.axis_index("c")
    s = lax.axis_index("s")
    wid = s * NC + c
    pltpu.sync_copy(dst_hbm.at[pl.ds(wid * EPT_P, EPT_P)], idx_v)

    ones = jnp.ones((16,), jnp.float32)
    zeros = jnp.zeros((16,), jnp.float32)

    @pl.loop(0, N_PAD // 16)
    def _zero(col):
        hist_v[pl.ds(col * 16, 16)] = zeros

    @pl.loop(0, EPT_P // 16)
    def _scan(e):
        idx16 = idx_v[pl.ds(e * 16, 16)]
        plsc.addupdate_scatter(hist_v, [idx16], ones)

    pltpu.sync_copy(hist_v, deg_part_hbm.at[wid])


_sc_degree = functools.partial(
    pl.kernel,
    out_type=jax.ShapeDtypeStruct((NW, N_PAD), jnp.float32),
    mesh=plsc.VectorSubcoreMesh(core_axis_name="c", subcore_axis_name="s",
                                num_cores=NC, num_subcores=NS),
    scratch_types=[
        pltpu.VMEM((N_PAD,), jnp.int32),
        pltpu.VMEM((N_PAD,), jnp.float32),
    ],
    compiler_params=pltpu.CompilerParams(needs_layout_passes=False),
)(_sc_degree_body)


def _tc_prep_body(deg_ref, data_ref, w_ref, y_ref, dinv_ref):
    deg = jnp.sum(deg_ref[...], axis=0)[:N_NODES] + 1.0
    dinv = lax.rsqrt(deg)[:, None]
    xw = jnp.dot(data_ref[...], w_ref[...],
                 preferred_element_type=jnp.float32)
    y_ref[...] = xw * dinv
    dinv_ref[...] = dinv


def _tc_prep(deg_part, data, W):
    return pl.pallas_call(
        _tc_prep_body,
        out_shape=[
            jax.ShapeDtypeStruct((N_NODES, D), jnp.float32),
            jax.ShapeDtypeStruct((N_NODES, 1), jnp.float32),
        ],
    )(deg_part, data, W)


G = 128
EPT_P = 10240
NCHUNK = EPT_P // G
E_PAD = NW * EPT_P


def _sc_scatter_body(src_hbm, dst_hbm, y_hbm, zeros_hbm, part_hbm,
                     sidx_a, sidx_b, didx_a, didx_b, rows_a, rows_b,
                     si_a, si_b, sd_a, sd_b, sg_a, sg_b, acc_sh):
    c = lax.axis_index("c")
    s = lax.axis_index("s")
    wid = s * NC + c
    base = wid * EPT_P

    pltpu.sync_copy(zeros_hbm, acc_sh.at[pl.ds(s * ROWS_PT, ROWS_PT), :])

    plsc.subcore_barrier()

    def start_i(i, sidx, didx, si, sd):
        pltpu.async_copy(src_hbm.at[pl.ds(base + i * G, G)], sidx, si)
        pltpu.async_copy(dst_hbm.at[pl.ds(base + i * G, G)], didx, sd)

    def wait_i(sidx, didx, si, sd):
        pltpu.make_async_copy(src_hbm.at[pl.ds(0, G)], sidx, si).wait()
        pltpu.make_async_copy(dst_hbm.at[pl.ds(0, G)], didx, sd).wait()

    def start_g(sidx, rows, sg):
        pltpu.async_copy(y_hbm.at[sidx], rows, sg)

    def wait_g(sidx, rows, sg):
        pltpu.make_async_copy(y_hbm.at[sidx], rows, sg).wait()

    start_i(0, sidx_a, didx_a, si_a, sd_a)
    start_i(1, sidx_b, didx_b, si_b, sd_b)
    wait_i(sidx_a, didx_a, si_a, sd_a)
    start_g(sidx_a, rows_a, sg_a)

    @pl.loop(0, NCHUNK, step=2)
    def _chunk(i):
        wait_i(sidx_b, didx_b, si_b, sd_b)
        start_g(sidx_b, rows_b, sg_b)
        wait_g(sidx_a, rows_a, sg_a)
        pltpu.sync_copy(rows_a, acc_sh.at[didx_a], add=True)
        start_i(i + 2, sidx_a, didx_a, si_a, sd_a)
        wait_i(sidx_a, didx_a, si_a, sd_a)
        start_g(sidx_a, rows_a, sg_a)
        wait_g(sidx_b, rows_b, sg_b)
        pltpu.sync_copy(rows_b, acc_sh.at[didx_b], add=True)
        start_i(i + 3, sidx_b, didx_b, si_b, sd_b)

    wait_g(sidx_a, rows_a, sg_a)
    wait_i(sidx_b, didx_b, si_b, sd_b)

    plsc.subcore_barrier()

    pltpu.sync_copy(acc_sh.at[pl.ds(s * ROWS_PT, ROWS_PT), :],
                    part_hbm.at[c, pl.ds(s * ROWS_PT, ROWS_PT), :])


_sc_scatter = functools.partial(
    pl.kernel,
    out_type=jax.ShapeDtypeStruct((NC, N_PAD, D), jnp.float32),
    mesh=plsc.VectorSubcoreMesh(core_axis_name="c", subcore_axis_name="s",
                                num_cores=NC, num_subcores=NS),
    scratch_types=[
        pltpu.VMEM((G,), jnp.int32),
        pltpu.VMEM((G,), jnp.int32),
        pltpu.VMEM((G,), jnp.int32),
        pltpu.VMEM((G,), jnp.int32),
        pltpu.VMEM((G, D), jnp.float32),
        pltpu.VMEM((G, D), jnp.float32),
        pltpu.SemaphoreType.DMA,
        pltpu.SemaphoreType.DMA,
        pltpu.SemaphoreType.DMA,
        pltpu.SemaphoreType.DMA,
        pltpu.SemaphoreType.DMA,
        pltpu.SemaphoreType.DMA,
        pltpu.VMEM_SHARED((N_PAD, D), jnp.float32),
    ],
)(_sc_scatter_body)


def _tc_finish_body(part_ref, y_ref, dinv_ref, b_ref, g_ref, beta_ref,
                    o_ref):
    s = part_ref[0, :N_NODES] + part_ref[1, :N_NODES] + y_ref[...]
    pre = s * dinv_ref[...] + b_ref[...]
    r = jnp.maximum(pre, 0.0)
    mean = jnp.mean(r, axis=0, keepdims=True)
    var = jnp.mean((r - mean) ** 2, axis=0, keepdims=True)
    o_ref[...] = (r - mean) / jnp.sqrt(var + 1e-5) * g_ref[...] + beta_ref[...]


def _tc_finish(part, y, dinv, b, g, beta):
    return pl.pallas_call(
        _tc_finish_body,
        out_shape=jax.ShapeDtypeStruct((N_NODES, D), jnp.float32),
    )(part, y, dinv, b, g, beta)


def kernel(data, edge_index, W, b, bn_gamma, bn_beta):
    src = edge_index[0].astype(jnp.int32)
    dst = edge_index[1].astype(jnp.int32)
    zeros_rows = jnp.zeros((ROWS_PT, D), jnp.float32)
    ppt = EPT_P - EPT
    pad_src = jnp.broadcast_to(
        (jnp.arange(ppt, dtype=jnp.int32) * 41) % N_NODES, (NW, ppt))
    pad_dst = jnp.broadcast_to(
        N_NODES + (jnp.arange(ppt, dtype=jnp.int32) % (N_PAD - N_NODES)),
        (NW, ppt))
    src_p = jnp.concatenate(
        [src.reshape(NW, EPT), pad_src], axis=1).reshape(-1)
    dst_p = jnp.concatenate(
        [dst.reshape(NW, EPT), pad_dst], axis=1).reshape(-1)
    tail = jnp.zeros((2 * G,), jnp.int32)
    src_p = jnp.concatenate([src_p, tail])
    dst_p = jnp.concatenate([dst_p, N_NODES + tail])
    deg_part = _sc_degree(dst_p)
    y, dinv = _tc_prep(deg_part, data, W)
    part = _sc_scatter(src_p, dst_p, y, zeros_rows)
    return _tc_finish(part, y, dinv, b.reshape(1, D),
                      bn_gamma.reshape(1, D), bn_beta.reshape(1, D))
